# baseline (speedup 1.0000x reference)
import functools
import math

import jax
import jax.numpy as jnp
from jax import lax
from jax.experimental import pallas as pl
from jax.experimental.pallas import tpu as pltpu
from jax.experimental.pallas import tpu_sc as plsc

B = 16384
DATA_DIM = 512
DIM = 64
NUM_CELL = 100
TWO_PI = 2.0 * math.pi


_BS = 1024


def _tc_body(time_ref, xt_ref, cell_ref, dose_ref, tf_ref, df_ref, wp_ref,
             bp_ref, ct_ref, time_out, xt_out, cell_out, dose_out):
    t = time_ref[...]
    proj = TWO_PI * t * tf_ref[...]
    time_out[...] = jnp.concatenate([jnp.sin(proj), jnp.cos(proj)], axis=-1)

    xt_out[...] = (
        jnp.dot(xt_ref[...], wp_ref[...], preferred_element_type=jnp.float32)
        + bp_ref[...]
    )

    idx = cell_ref[...]
    iota = lax.broadcasted_iota(jnp.int32, (_BS, NUM_CELL), 1)
    onehot = (iota == idx).astype(jnp.float32)
    cell_out[...] = jnp.dot(onehot, ct_ref[...],
                            preferred_element_type=jnp.float32)

    d = dose_ref[...]
    dproj = TWO_PI * d * df_ref[...]
    dose_out[...] = jnp.concatenate([jnp.sin(dproj), jnp.cos(dproj)], axis=-1)


def _tc_dense(time, xt, cell_type, dose_flat, time_freqs, dose_freqs, Wp, bp,
              cell_table):
    half = DIM // 2
    grid = (B // _BS,)
    return pl.pallas_call(
        _tc_body,
        grid=grid,
        in_specs=[
            pl.BlockSpec((_BS, 1), lambda i: (i, 0)),
            pl.BlockSpec((_BS, DATA_DIM), lambda i: (i, 0)),
            pl.BlockSpec((_BS, 1), lambda i: (i, 0)),
            pl.BlockSpec((3 * _BS, 1), lambda i: (i, 0)),
            pl.BlockSpec((1, half), lambda i: (0, 0)),
            pl.BlockSpec((1, half), lambda i: (0, 0)),
            pl.BlockSpec((DATA_DIM, DIM), lambda i: (0, 0)),
            pl.BlockSpec((1, DIM), lambda i: (0, 0)),
            pl.BlockSpec((NUM_CELL, DIM), lambda i: (0, 0)),
        ],
        out_specs=[
            pl.BlockSpec((_BS, DIM), lambda i: (i, 0)),
            pl.BlockSpec((_BS, DIM), lambda i: (i, 0)),
            pl.BlockSpec((_BS, DIM), lambda i: (i, 0)),
            pl.BlockSpec((3 * _BS, DIM), lambda i: (i, 0)),
        ],
        out_shape=[
            jax.ShapeDtypeStruct((B, DIM), jnp.float32),
            jax.ShapeDtypeStruct((B, DIM), jnp.float32),
            jax.ShapeDtypeStruct((B, DIM), jnp.float32),
            jax.ShapeDtypeStruct((3 * B, DIM), jnp.float32),
        ],
    )(
        time.reshape(B, 1), xt, cell_type.reshape(B, 1),
        dose_flat.reshape(3 * B, 1), time_freqs.reshape(1, half),
        dose_freqs.reshape(1, half), Wp, bp.reshape(1, DIM), cell_table,
    )


_CHUNK = 128


def _sc_gather(table, idx_flat):
    n = idx_flat.shape[0]
    d = table.shape[1]
    info = plsc.get_sparse_core_info()
    nw = info.num_cores * info.num_subcores
    per_w = n // nw
    n_ch = per_w // _CHUNK
    assert per_w % _CHUNK == 0
    idx3d = idx_flat.reshape(nw, n_ch, _CHUNK)
    mesh = plsc.VectorSubcoreMesh(core_axis_name="c", subcore_axis_name="s")

    def body(idx_hbm, table_hbm, out_hbm, idx_v, rows_v, sem):
        wid = lax.axis_index("s") * info.num_cores + lax.axis_index("c")
        pltpu.sync_copy(idx_hbm.at[wid], idx_v)
        for j in range(n_ch):
            pltpu.async_copy(table_hbm.at[idx_v.at[j]], rows_v, sem).wait()
            pltpu.sync_copy(
                rows_v, out_hbm.at[pl.ds(wid * per_w + j * _CHUNK, _CHUNK)])

    fn = pl.kernel(
        body,
        out_type=jax.ShapeDtypeStruct((n, d), jnp.float32),
        mesh=mesh,
        scratch_types=[
            pltpu.VMEM((n_ch, _CHUNK), jnp.int32),
            pltpu.VMEM((_CHUNK, d), jnp.float32),
            pltpu.SemaphoreType.DMA,
        ],
        compiler_params=pltpu.CompilerParams(use_tc_tiling_on_sc=False),
    )
    return fn(idx3d, table)


def kernel(time, xt, cell_type, gene_pert_idx, mol_pert_idx, mol_dose,
           time_freqs, dose_freqs, Wp, bp, cell_table, gene_table, mol_table):
    time_emb, xt_emb, cell_emb, dose_flat_emb = _tc_dense(
        time, xt, cell_type, mol_dose.reshape(-1), time_freqs, dose_freqs,
        Wp, bp, cell_table)

    gene_flat = _sc_gather(gene_table, gene_pert_idx.reshape(-1))
    mol_flat = _sc_gather(mol_table, mol_pert_idx.reshape(-1))

    gene_emb = gene_flat.reshape(3, B, DIM)
    mol_emb = mol_flat.reshape(3, B, DIM)
    dose_emb = dose_flat_emb.reshape(3, B, DIM)
    return (time_emb, xt_emb, cell_emb, gene_emb, mol_emb, dose_emb)


# speedup vs baseline: 1.1881x; 1.1881x over previous
import functools
import math

import jax
import jax.numpy as jnp
from jax import lax
from jax.experimental import pallas as pl
from jax.experimental.pallas import tpu as pltpu
from jax.experimental.pallas import tpu_sc as plsc

B = 16384
DATA_DIM = 512
DIM = 64
HALF = DIM // 2
NUM_CELL = 100

_SIN_C = (6.283168273564918, -41.337929774906165, 81.47313282270473,
          -75.0932740471627, 33.95650071282797)
_COS_C = (0.9999994434755294, -19.739034355263385, 64.93061294590028,
          -85.29596684284616, 58.91253793524945, -21.282995036331283)


def _sincos_2pi(u):
    r = u - jnp.floor(u + 0.5)
    z = r * r
    s = _SIN_C[4]
    for c in (_SIN_C[3], _SIN_C[2], _SIN_C[1], _SIN_C[0]):
        s = s * z + c
    s = s * r
    q = _COS_C[5]
    for c in (_COS_C[4], _COS_C[3], _COS_C[2], _COS_C[1], _COS_C[0]):
        q = q * z + c
    return s, q


_BS = 2048


def _tc_body(t_ref, xt_ref, cell_ref, dose_ref, tf_ref, df_ref, wp_ref,
             bp_ref, ct_ref, time_out, xt_out, cell_out, dose_out):
    u = tf_ref[...] * t_ref[...]
    s, c = _sincos_2pi(u)
    time_out[0:HALF, :] = s
    time_out[HALF:DIM, :] = c

    xt_out[...] = lax.dot_general(
        wp_ref[...], xt_ref[...], (((0,), (1,)), ((), ())),
        preferred_element_type=jnp.float32) + bp_ref[...]

    idx = cell_ref[...]
    iota = lax.broadcasted_iota(jnp.int32, (NUM_CELL, _BS), 0)
    onehot = (iota == idx).astype(jnp.float32)
    cell_out[...] = lax.dot_general(
        ct_ref[...], onehot, (((0,), (0,)), ((), ())),
        preferred_element_type=jnp.float32)

    for p in range(3):
        up = df_ref[...] * dose_ref[pl.ds(p, 1), :]
        sp, cp = _sincos_2pi(up)
        dose_out[pl.ds(DIM * p, HALF), :] = sp
        dose_out[pl.ds(DIM * p + HALF, HALF), :] = cp


def _tc_dense(time, xt, cell_type, dose3, time_freqs, dose_freqs, Wp, bp,
              cell_table):
    grid = (B // _BS,)
    return pl.pallas_call(
        _tc_body,
        grid=grid,
        in_specs=[
            pl.BlockSpec((1, _BS), lambda i: (0, i)),
            pl.BlockSpec((_BS, DATA_DIM), lambda i: (i, 0)),
            pl.BlockSpec((1, _BS), lambda i: (0, i)),
            pl.BlockSpec((3, _BS), lambda i: (0, i)),
            pl.BlockSpec((HALF, 1), lambda i: (0, 0)),
            pl.BlockSpec((HALF, 1), lambda i: (0, 0)),
            pl.BlockSpec((DATA_DIM, DIM), lambda i: (0, 0)),
            pl.BlockSpec((DIM, 1), lambda i: (0, 0)),
            pl.BlockSpec((NUM_CELL, DIM), lambda i: (0, 0)),
        ],
        out_specs=[
            pl.BlockSpec((DIM, _BS), lambda i: (0, i)),
            pl.BlockSpec((DIM, _BS), lambda i: (0, i)),
            pl.BlockSpec((DIM, _BS), lambda i: (0, i)),
            pl.BlockSpec((3 * DIM, _BS), lambda i: (0, i)),
        ],
        out_shape=[
            jax.ShapeDtypeStruct((DIM, B), jnp.float32),
            jax.ShapeDtypeStruct((DIM, B), jnp.float32),
            jax.ShapeDtypeStruct((DIM, B), jnp.float32),
            jax.ShapeDtypeStruct((3 * DIM, B), jnp.float32),
        ],
    )(
        time.reshape(1, B), xt, cell_type.reshape(1, B), dose3,
        time_freqs.reshape(HALF, 1), dose_freqs.reshape(HALF, 1), Wp,
        bp.reshape(DIM, 1), cell_table,
    )


_CHUNK = 128


def _sc_gather(table, idx_flat):
    n = idx_flat.shape[0]
    d = table.shape[1]
    info = plsc.get_sparse_core_info()
    nw = info.num_cores * info.num_subcores
    per_w = n // nw
    n_ch = per_w // _CHUNK
    assert per_w % _CHUNK == 0
    idx3d = idx_flat.reshape(nw, n_ch, _CHUNK)
    mesh = plsc.VectorSubcoreMesh(core_axis_name="c", subcore_axis_name="s")

    def body(idx_hbm, table_hbm, out_hbm, idx_v, rows_v, sem):
        wid = lax.axis_index("s") * info.num_cores + lax.axis_index("c")
        pltpu.sync_copy(idx_hbm.at[wid], idx_v)
        for j in range(n_ch):
            pltpu.async_copy(table_hbm.at[idx_v.at[j]], rows_v, sem).wait()
            pltpu.sync_copy(
                rows_v, out_hbm.at[pl.ds(wid * per_w + j * _CHUNK, _CHUNK)])

    fn = pl.kernel(
        body,
        out_type=jax.ShapeDtypeStruct((n, d), jnp.float32),
        mesh=mesh,
        scratch_types=[
            pltpu.VMEM((n_ch, _CHUNK), jnp.int32),
            pltpu.VMEM((_CHUNK, d), jnp.float32),
            pltpu.SemaphoreType.DMA,
        ],
        compiler_params=pltpu.CompilerParams(use_tc_tiling_on_sc=False),
    )
    return fn(idx3d, table)


def kernel(time, xt, cell_type, gene_pert_idx, mol_pert_idx, mol_dose,
           time_freqs, dose_freqs, Wp, bp, cell_table, gene_table, mol_table):
    dose3 = mol_dose.reshape(3, B)
    time_T, xt_T, cell_T, dose_T = _tc_dense(
        time, xt, cell_type, dose3, time_freqs, dose_freqs, Wp, bp,
        cell_table)

    gene_flat = _sc_gather(gene_table, gene_pert_idx.reshape(-1))
    mol_flat = _sc_gather(mol_table, mol_pert_idx.reshape(-1))

    time_emb = time_T.T
    xt_emb = xt_T.T
    cell_emb = cell_T.T
    dose_emb = dose_T.reshape(3, DIM, B).transpose(0, 2, 1)
    gene_emb = gene_flat.reshape(3, B, DIM)
    mol_emb = mol_flat.reshape(3, B, DIM)
    return (time_emb, xt_emb, cell_emb, gene_emb, mol_emb, dose_emb)
